# baseline (speedup 1.0000x reference)
import functools

import jax
import jax.numpy as jnp
from jax import lax
from jax.experimental import pallas as pl
from jax.experimental.pallas import tpu as pltpu
from jax.experimental.pallas import tpu_sc as plsc

_N = 10000
_E = 320000
_D = 128

_CHUNK = 128
_NCH = _E // _CHUNK
_CPW = _NCH // 32
_XTRA = _NCH - 32 * _CPW
_ROWS_A = 624
_ROWS_LAST = _N - 15 * _ROWS_A


def _sc_segment_sum(edge_attr, edge_index, zeros):
  mesh = plsc.VectorSubcoreMesh(core_axis_name="c", subcore_axis_name="s")

  @functools.partial(
      pl.kernel,
      out_type=jax.ShapeDtypeStruct((2, _N, _D), jnp.float32),
      mesh=mesh,
      scratch_types=[
          pltpu.VMEM((3, _CHUNK, _D), jnp.float32),
          pltpu.VMEM((3, 2, _CHUNK), jnp.int32),
          pltpu.VMEM_SHARED((_N, _D), jnp.float32),
      ] + [pltpu.SemaphoreType.DMA] * 9,
  )
  def k(ea_hbm, col_hbm, z_hbm, out_hbm, rows_v, idx_v, acc_sh, *sems):
    cid = lax.axis_index("c")
    sid = lax.axis_index("s")
    r0 = pl.multiple_of(sid * _ROWS_A, 8)

    wid = sid * 2 + cid
    cbase = wid * _CPW
    gs = sems[0:3]
    isx = sems[3:6]
    ss = sems[6:9]

    def issue_gather(ci, b):
      base = pl.multiple_of(ci * _CHUNK, _CHUNK)
      pltpu.async_copy(ea_hbm.at[pl.ds(base, _CHUNK)], rows_v.at[b], gs[b])
      pltpu.async_copy(col_hbm.at[pl.ds(0, 2), pl.ds(base, _CHUNK)],
                       idx_v.at[b], isx[b])

    def wait_gather(b):
      pltpu.make_async_copy(ea_hbm.at[pl.ds(0, _CHUNK)],
                            rows_v.at[b], gs[b]).wait()
      pltpu.make_async_copy(col_hbm.at[pl.ds(0, 2), pl.ds(0, _CHUNK)],
                            idx_v.at[b], isx[b]).wait()

    def issue_scatter(b):
      pltpu.async_copy(rows_v.at[b], acc_sh.at[idx_v.at[b, 1]], ss[b],
                       add=True)

    def wait_scatter(b):
      pltpu.make_async_copy(rows_v.at[b], acc_sh.at[idx_v.at[b, 1]],
                            ss[b]).wait()

    def slot(ci, b, prefetch=True, wscatter=True):
      b2 = (b + 2) % 3
      wait_gather(b)
      if wscatter:
        wait_scatter(b2)
      issue_scatter(b)
      if prefetch:
        issue_gather(ci + 2, b2)

    issue_gather(cbase, 0)
    issue_gather(cbase + 1, 1)

    @pl.when(sid < 15)
    def _():
      pltpu.sync_copy(z_hbm.at[pl.ds(0, _ROWS_A)],
                      acc_sh.at[pl.ds(r0, _ROWS_A)])

    @pl.when(sid == 15)
    def _():
      pltpu.sync_copy(z_hbm, acc_sh.at[pl.ds(15 * _ROWS_A, _ROWS_LAST)])

    plsc.subcore_barrier()

    slot(cbase, 0, wscatter=False)
    slot(cbase + 1, 1)

    def tri(p, carry):
      c0 = cbase + 3 * p + 2
      for j, b in enumerate((2, 0, 1)):
        slot(c0 + j, b)
      return carry

    lax.fori_loop(0, (_CPW - 2) // 3, tri, 0)
    slot(cbase + _CPW - 1, (_CPW - 1) % 3, prefetch=False)
    wait_gather(_CPW % 3)

    @pl.when(wid < _XTRA)
    def _():
      b = _CPW % 3
      issue_gather(32 * _CPW + wid, b)
      wait_gather(b)
      wait_scatter((_CPW - 1) % 3)
      issue_scatter(b)
      wait_scatter(b)

    @pl.when(wid >= _XTRA)
    def _():
      wait_scatter((_CPW - 1) % 3)

    plsc.subcore_barrier()

    @pl.when(sid < 15)
    def _():
      pltpu.sync_copy(acc_sh.at[pl.ds(r0, _ROWS_A)],
                      out_hbm.at[cid, pl.ds(r0, _ROWS_A)])

    @pl.when(sid == 15)
    def _():
      pltpu.sync_copy(acc_sh.at[pl.ds(15 * _ROWS_A, _ROWS_LAST)],
                      out_hbm.at[cid, pl.ds(15 * _ROWS_A, _ROWS_LAST)])

  return k(edge_attr, edge_index, zeros)


_R = 1000


def _dott(a, w):
  return lax.dot_general(a, w, (((1,), (1,)), ((), ())),
                         preferred_element_type=jnp.float32)


def _tc_mlp(x, parts, W0, b0, W1, b1, W2, b2, W3, b3, ln_g, ln_b):
  def body(x_ref, p0_ref, p1_ref, w0_ref, w1_ref, w2_ref, w3_ref,
           b0_ref, b1_ref, b2_ref, b3_ref, g_ref, bb_ref, o_ref):
    agg = p0_ref[0] + p1_ref[0]
    w0 = w0_ref[...]
    h = _dott(x_ref[...], w0[:, :_D]) + _dott(agg, w0[:, _D:])
    h = jnp.maximum(h + b0_ref[...], 0.0)
    h = jnp.maximum(_dott(h, w1_ref[...]) + b1_ref[...], 0.0)
    h = jnp.maximum(_dott(h, w2_ref[...]) + b2_ref[...], 0.0)
    h = _dott(h, w3_ref[...]) + b3_ref[...]
    mu = jnp.mean(h, axis=-1, keepdims=True)
    c = h - mu
    var = jnp.mean(c * c, axis=-1, keepdims=True)
    o_ref[...] = g_ref[...] * c * lax.rsqrt(var + 1e-5) + bb_ref[...]

  def rows(nc):
    return pl.BlockSpec((_R, nc), lambda i: (i, 0))

  def part(j):
    return pl.BlockSpec((1, _R, _D), lambda i, j=j: (j, i, 0))

  def full(s):
    return pl.BlockSpec(s, lambda i: (0,) * len(s))

  return pl.pallas_call(
      body,
      grid=(_N // _R,),
      in_specs=[rows(_D), part(0), part(1),
                full((256, 256)), full((256, 256)), full((256, 256)),
                full((_D, 256)),
                full((1, 256)), full((1, 256)), full((1, 256)),
                full((1, _D)), full((1, _D)), full((1, _D))],
      out_specs=rows(_D),
      out_shape=jax.ShapeDtypeStruct((_N, _D), jnp.float32),
  )(x, parts, parts, W0, W1, W2, W3,
    b0[None, :], b1[None, :], b2[None, :], b3[None, :],
    ln_g[None, :], ln_b[None, :])


def kernel(x, edge_index, edge_attr, u, batch,
           W0, b0, W1, b1, W2, b2, W3, b3, ln_g, ln_b):
  del u, batch
  zeros = jnp.zeros((_ROWS_LAST, _D), jnp.float32)
  parts = _sc_segment_sum(edge_attr, edge_index, zeros)
  return _tc_mlp(x, parts, W0, b0, W1, b1, W2, b2, W3, b3, ln_g, ln_b)


# speedup vs baseline: 1.0253x; 1.0253x over previous
import functools

import jax
import jax.numpy as jnp
from jax import lax
from jax.experimental import pallas as pl
from jax.experimental.pallas import tpu as pltpu
from jax.experimental.pallas import tpu_sc as plsc

_N = 10000
_E = 320000
_D = 128

_CHUNK = 128
_NCH = _E // _CHUNK
_CPW = _NCH // 32
_XTRA = _NCH - 32 * _CPW
_ROWS_A = 624
_ROWS_LAST = _N - 15 * _ROWS_A


def _sc_segment_sum(edge_attr, edge_index, zeros):
  mesh = plsc.VectorSubcoreMesh(core_axis_name="c", subcore_axis_name="s")

  @functools.partial(
      pl.kernel,
      out_type=jax.ShapeDtypeStruct((2, _N, _D), jnp.float32),
      mesh=mesh,
      scratch_types=[
          pltpu.VMEM((3, _CHUNK, _D), jnp.float32),
          pltpu.VMEM((3, 2, _CHUNK), jnp.int32),
          pltpu.VMEM_SHARED((_N, _D), jnp.float32),
      ] + [pltpu.SemaphoreType.DMA] * 9,
  )
  def k(ea_hbm, col_hbm, z_hbm, out_hbm, rows_v, idx_v, acc_sh, *sems):
    cid = lax.axis_index("c")
    sid = lax.axis_index("s")
    r0 = pl.multiple_of(sid * _ROWS_A, 8)

    wid = sid * 2 + cid
    cbase = wid * _CPW
    gs = sems[0:3]
    isx = sems[3:6]
    ss = sems[6:9]

    def issue_gather(ci, b):
      base = pl.multiple_of(ci * _CHUNK, _CHUNK)
      pltpu.async_copy(ea_hbm.at[pl.ds(base, _CHUNK)], rows_v.at[b], gs[b])
      pltpu.async_copy(col_hbm.at[pl.ds(0, 2), pl.ds(base, _CHUNK)],
                       idx_v.at[b], isx[b])

    def wait_gather(b):
      pltpu.make_async_copy(ea_hbm.at[pl.ds(0, _CHUNK)],
                            rows_v.at[b], gs[b]).wait()
      pltpu.make_async_copy(col_hbm.at[pl.ds(0, 2), pl.ds(0, _CHUNK)],
                            idx_v.at[b], isx[b]).wait()

    def issue_scatter(b):
      pltpu.async_copy(rows_v.at[b], acc_sh.at[idx_v.at[b, 1]], ss[b],
                       add=True)

    def wait_scatter(b):
      pltpu.make_async_copy(rows_v.at[b], acc_sh.at[idx_v.at[b, 1]],
                            ss[b]).wait()

    def slot(ci, b, prefetch=True, wscatter=True):
      b2 = (b + 2) % 3
      wait_gather(b)
      if wscatter:
        wait_scatter(b2)
      issue_scatter(b)
      if prefetch:
        issue_gather(ci + 2, b2)

    issue_gather(cbase, 0)
    issue_gather(cbase + 1, 1)

    @pl.when(sid < 15)
    def _():
      pltpu.sync_copy(z_hbm.at[pl.ds(0, _ROWS_A)],
                      acc_sh.at[pl.ds(r0, _ROWS_A)])

    @pl.when(sid == 15)
    def _():
      pltpu.sync_copy(z_hbm, acc_sh.at[pl.ds(15 * _ROWS_A, _ROWS_LAST)])

    plsc.subcore_barrier()

    slot(cbase, 0, wscatter=False)
    slot(cbase + 1, 1)

    def tri(p, carry):
      c0 = cbase + 3 * p + 2
      for j, b in enumerate((2, 0, 1)):
        slot(c0 + j, b)
      return carry

    lax.fori_loop(0, (_CPW - 2) // 3, tri, 0)
    slot(cbase + _CPW - 1, (_CPW - 1) % 3, prefetch=False)
    wait_gather(_CPW % 3)

    @pl.when(wid < _XTRA)
    def _():
      b = _CPW % 3
      issue_gather(32 * _CPW + wid, b)
      wait_gather(b)
      wait_scatter((_CPW - 1) % 3)
      issue_scatter(b)
      wait_scatter(b)

    @pl.when(wid >= _XTRA)
    def _():
      wait_scatter((_CPW - 1) % 3)

    plsc.subcore_barrier()

    @pl.when(sid < 15)
    def _():
      pltpu.sync_copy(acc_sh.at[pl.ds(r0, _ROWS_A)],
                      out_hbm.at[cid, pl.ds(r0, _ROWS_A)])

    @pl.when(sid == 15)
    def _():
      pltpu.sync_copy(acc_sh.at[pl.ds(15 * _ROWS_A, _ROWS_LAST)],
                      out_hbm.at[cid, pl.ds(15 * _ROWS_A, _ROWS_LAST)])

  return k(edge_attr, edge_index, zeros)


_R = 2000


def _dott(a, w):
  return lax.dot_general(a, w, (((1,), (1,)), ((), ())),
                         preferred_element_type=jnp.float32)


def _tc_mlp(x, parts, W0, b0, W1, b1, W2, b2, W3, b3, ln_g, ln_b):
  def body(x_ref, p0_ref, p1_ref, w0_ref, w1_ref, w2_ref, w3_ref,
           b0_ref, b1_ref, b2_ref, b3_ref, g_ref, bb_ref, o_ref):
    agg = p0_ref[0] + p1_ref[0]
    w0 = w0_ref[...]
    h = _dott(x_ref[...], w0[:, :_D]) + _dott(agg, w0[:, _D:])
    h = jnp.maximum(h + b0_ref[...], 0.0)
    h = jnp.maximum(_dott(h, w1_ref[...]) + b1_ref[...], 0.0)
    h = jnp.maximum(_dott(h, w2_ref[...]) + b2_ref[...], 0.0)
    h = _dott(h, w3_ref[...]) + b3_ref[...]
    mu = jnp.mean(h, axis=-1, keepdims=True)
    c = h - mu
    var = jnp.mean(c * c, axis=-1, keepdims=True)
    o_ref[...] = g_ref[...] * c * lax.rsqrt(var + 1e-5) + bb_ref[...]

  def rows(nc):
    return pl.BlockSpec((_R, nc), lambda i: (i, 0))

  def part(j):
    return pl.BlockSpec((1, _R, _D), lambda i, j=j: (j, i, 0))

  def full(s):
    return pl.BlockSpec(s, lambda i: (0,) * len(s))

  return pl.pallas_call(
      body,
      grid=(_N // _R,),
      in_specs=[rows(_D), part(0), part(1),
                full((256, 256)), full((256, 256)), full((256, 256)),
                full((_D, 256)),
                full((1, 256)), full((1, 256)), full((1, 256)),
                full((1, _D)), full((1, _D)), full((1, _D))],
      out_specs=rows(_D),
      out_shape=jax.ShapeDtypeStruct((_N, _D), jnp.float32),
  )(x, parts, parts, W0, W1, W2, W3,
    b0[None, :], b1[None, :], b2[None, :], b3[None, :],
    ln_g[None, :], ln_b[None, :])


def kernel(x, edge_index, edge_attr, u, batch,
           W0, b0, W1, b1, W2, b2, W3, b3, ln_g, ln_b):
  del u, batch
  zeros = jnp.zeros((_ROWS_LAST, _D), jnp.float32)
  parts = _sc_segment_sum(edge_attr, edge_index, zeros)
  return _tc_mlp(x, parts, W0, b0, W1, b1, W2, b2, W3, b3, ln_g, ln_b)
